# Initial kernel scaffold; baseline (speedup 1.0000x reference)
import jax
import jax.numpy as jnp
from jax.experimental import pallas as pl


def kernel(xyz, points, center, t_list, W1, b1, g1, be1, W2, b2, g2, be2):
    raise NotImplementedError("write your pallas kernel here")


# speedup vs baseline: 8.9838x; 8.9838x over previous
import functools

import jax
import jax.numpy as jnp
from jax import lax
from jax.experimental import pallas as pl
from jax.experimental.pallas import tpu as pltpu
from jax.experimental.pallas import tpu_sc as plsc

_RADIUS = 2.0
_NS = 32
_W_LIST = ((1.0, 0.2), (0.5, 1.0))
_BIG = 1.0e9

_SBLK_SEL = 128
_SBLK_MLP = 128

_SC_NC = 2
_SC_NSUB = 16
_SC_NW = _SC_NC * _SC_NSUB
_SC_CB = 128


def _sel_body(tl_ref, xyzT_ref, cen_ref, idx1_ref, idx2_ref, *, n, sblk):
    b = pl.program_id(0)
    cen = cen_ref[0]
    pts = xyzT_ref[0]
    tl = tl_ref[0]
    gaps = tl[:, 1:] - tl[:, :-1]
    mt = jnp.max(gaps) ** 2

    cen_b = cen.astype(jnp.bfloat16).astype(jnp.float32)
    pts_b = pts.astype(jnp.bfloat16).astype(jnp.float32)
    sn_xy = jnp.sum(cen[:, 0:2] ** 2, axis=1, keepdims=True)
    dn_xy = jnp.sum(pts[0:2, :] ** 2, axis=0, keepdims=True)
    mm_xy = cen_b[:, 0:1] * pts_b[0:1, :] + cen_b[:, 1:2] * pts_b[1:2, :]
    dxy = -2.0 * mm_xy
    dxy = dxy + sn_xy
    dxy = dxy + dn_xy
    st = cen[:, 2:3]
    dtp = pts[2:3, :]
    dt = -2.0 * (st * dtp)
    dt = dt + st ** 2
    dt = dt + dtp ** 2
    mm_o = (cen_b[:, 0:1] * pts_b[0:1, :] + cen_b[:, 1:2] * pts_b[1:2, :]) \
        + cen_b[:, 2:3] * pts_b[2:3, :]
    dori = -2.0 * mm_o
    dori = dori + jnp.sum(cen ** 2, axis=1, keepdims=True)
    dori = dori + jnp.sum(pts ** 2, axis=0, keepdims=True)
    valid = (dxy <= _RADIUS) & (dt <= mt)
    iota = lax.broadcasted_iota(jnp.int32, (sblk, n), 1)

    m0 = jnp.min(dori, axis=1, keepdims=True)
    gf = jnp.min(jnp.where(dori == m0, iota, n), axis=1, keepdims=True)

    kiota = lax.broadcasted_iota(jnp.int32, (sblk, _NS), 1)
    base = b * n

    for wi, (w0, w1) in enumerate(_W_LIST):
        d0 = jnp.where(valid, dxy * w0 + dt * w1, _BIG)
        acc0 = jnp.zeros((sblk, _NS), jnp.int32)

        def body(k, carry):
            d, acc = carry
            mv = jnp.min(d, axis=1, keepdims=True)
            ai = jnp.min(jnp.where(d == mv, iota, n), axis=1, keepdims=True)
            sel = jnp.where(mv >= _BIG, gf, ai)
            acc = jnp.where(kiota == k, sel + base, acc)
            d = jnp.where(iota == ai, _BIG, d)
            return d, acc

        _, acc = lax.fori_loop(0, _NS, body, (d0, acc0))
        if wi == 0:
            idx1_ref[0] = acc
        else:
            idx2_ref[0] = acc


def _select_indices(t_list, xyzT, center, interpret=False):
    B, _, N = xyzT.shape
    S = center.shape[1]
    grid = (B, S // _SBLK_SEL)
    out_shape = jax.ShapeDtypeStruct((B, S, _NS), jnp.int32)
    return pl.pallas_call(
        functools.partial(_sel_body, n=N, sblk=_SBLK_SEL),
        grid=grid,
        in_specs=[
            pl.BlockSpec((1, 1, 16), lambda b, s: (b, 0, 0)),
            pl.BlockSpec((1, 3, N), lambda b, s: (b, 0, 0)),
            pl.BlockSpec((1, _SBLK_SEL, 3), lambda b, s: (b, s, 0)),
        ],
        out_specs=[
            pl.BlockSpec((1, _SBLK_SEL, _NS), lambda b, s: (b, s, 0)),
            pl.BlockSpec((1, _SBLK_SEL, _NS), lambda b, s: (b, s, 0)),
        ],
        out_shape=[out_shape, out_shape],
        interpret=interpret,
    )(t_list, xyzT, center)


def _make_sc_gather(n_rows, dp):
    b_per_w = n_rows // _SC_NW
    n_chunks = b_per_w // _SC_CB
    mesh = plsc.VectorSubcoreMesh(core_axis_name="c", subcore_axis_name="s")

    @functools.partial(
        pl.kernel,
        mesh=mesh,
        compiler_params=pltpu.CompilerParams(use_tc_tiling_on_sc=False),
        out_type=jax.ShapeDtypeStruct((n_rows, dp), jnp.float32),
        scratch_types=[
            pltpu.VMEM((_SC_CB,), jnp.int32),
            pltpu.VMEM((_SC_CB, dp), jnp.float32),
            pltpu.SemaphoreType.DMA,
        ],
    )
    def gather_k(table_hbm, idx_hbm, out_hbm, idx_v, rows_v, sem):
        wid = lax.axis_index("s") * _SC_NC + lax.axis_index("c")
        base_w = wid * b_per_w

        def chunk(i, _):
            base = base_w + i * _SC_CB
            pltpu.sync_copy(idx_hbm.at[pl.ds(base, _SC_CB)], idx_v)
            pltpu.async_copy(table_hbm.at[idx_v], rows_v, sem).wait()
            pltpu.sync_copy(rows_v, out_hbm.at[pl.ds(base, _SC_CB)])
            return 0

        lax.fori_loop(0, n_chunks, chunk, 0)

    return gather_k


def _p1_body(g_ref, cen_ref, w1_ref, b1_ref, y1_ref, acc_ref, *, sblk):
    first = (pl.program_id(1) == 0) & (pl.program_id(2) == 0)
    x = g_ref[0, 0]
    cen = cen_ref[0]
    sub = jnp.concatenate(
        [cen, jnp.zeros((sblk, 29), jnp.float32)], axis=1)
    x3 = x.reshape(sblk, _NS, 32) - sub[:, None, :]
    xf = x3.reshape(sblk * _NS, 32)
    y1 = jnp.dot(xf, w1_ref[...], preferred_element_type=jnp.float32)
    y1 = y1 + b1_ref[...]
    y1_ref[0, 0] = y1

    s1 = jnp.sum(y1, axis=0, keepdims=True)
    s2 = jnp.sum(y1 * y1, axis=0, keepdims=True)
    z96 = jnp.zeros((1, 96), jnp.float32)
    row0 = jnp.concatenate([s1, z96], axis=1)
    row1 = jnp.concatenate([s2, z96], axis=1)
    blk = jnp.concatenate(
        [row0, row1, jnp.zeros((6, 128), jnp.float32)], axis=0)

    @pl.when(first)
    def _():
        acc_ref[...] = jnp.zeros_like(acc_ref)

    acc_ref[...] = acc_ref[...] + blk[None]


def _mlp_pass1(g4, center, w1pT, b1p, interpret=False):
    W, B, SN, _ = g4.shape
    S = center.shape[1]
    grid = (W, B, S // _SBLK_MLP)
    return pl.pallas_call(
        functools.partial(_p1_body, sblk=_SBLK_MLP),
        grid=grid,
        in_specs=[
            pl.BlockSpec((1, 1, _SBLK_MLP * _NS, 32), lambda w, b, s: (w, b, s, 0)),
            pl.BlockSpec((1, _SBLK_MLP, 3), lambda w, b, s: (b, s, 0)),
            pl.BlockSpec((32, 32), lambda w, b, s: (0, 0)),
            pl.BlockSpec((1, 32), lambda w, b, s: (0, 0)),
        ],
        out_specs=[
            pl.BlockSpec((1, 1, _SBLK_MLP * _NS, 32), lambda w, b, s: (w, b, s, 0)),
            pl.BlockSpec((1, 8, 128), lambda w, b, s: (w, 0, 0)),
        ],
        out_shape=[
            jax.ShapeDtypeStruct((W, B, SN, 32), jnp.float32),
            jax.ShapeDtypeStruct((W, 8, 128), jnp.float32),
        ],
        interpret=interpret,
    )(g4, center, w1pT, b1p)


def _p2_body(y1_ref, acc1_ref, g1_ref, be1_ref, w2_ref, b2_ref,
             ymax_ref, ymin_ref, acc2_ref, *, sblk, m_count):
    first = (pl.program_id(1) == 0) & (pl.program_id(2) == 0)
    inv = 1.0 / float(m_count)
    s1 = acc1_ref[0, 0:1, 0:32]
    s2 = acc1_ref[0, 1:2, 0:32]
    mu = s1 * inv
    var = s2 * inv - mu * mu
    a1 = g1_ref[...] * lax.rsqrt(var + 1e-5)
    c1 = be1_ref[...] - mu * a1

    y1 = y1_ref[0, 0]
    r = jnp.maximum(y1 * a1 + c1, 0.0)
    y2 = jnp.dot(r, w2_ref[...], preferred_element_type=jnp.float32)
    y2 = y2 + b2_ref[...]

    y3 = y2.reshape(sblk, _NS, 64)
    ymax_ref[0, 0] = jnp.max(y3, axis=1)
    ymin_ref[0, 0] = jnp.min(y3, axis=1)

    t1 = jnp.sum(y2, axis=0, keepdims=True)
    t2 = jnp.sum(y2 * y2, axis=0, keepdims=True)
    z64 = jnp.zeros((1, 64), jnp.float32)
    row0 = jnp.concatenate([t1, z64], axis=1)
    row1 = jnp.concatenate([t2, z64], axis=1)
    blk = jnp.concatenate(
        [row0, row1, jnp.zeros((6, 128), jnp.float32)], axis=0)

    @pl.when(first)
    def _():
        acc2_ref[...] = jnp.zeros_like(acc2_ref)

    acc2_ref[...] = acc2_ref[...] + blk[None]


def _mlp_pass2(y1, acc1, g1p, be1p, w2T, b2p, m_count, interpret=False):
    W, B, SN, _ = y1.shape
    S = SN // _NS
    grid = (W, B, S // _SBLK_MLP)
    return pl.pallas_call(
        functools.partial(_p2_body, sblk=_SBLK_MLP, m_count=m_count),
        grid=grid,
        in_specs=[
            pl.BlockSpec((1, 1, _SBLK_MLP * _NS, 32), lambda w, b, s: (w, b, s, 0)),
            pl.BlockSpec((1, 8, 128), lambda w, b, s: (w, 0, 0)),
            pl.BlockSpec((1, 32), lambda w, b, s: (0, 0)),
            pl.BlockSpec((1, 32), lambda w, b, s: (0, 0)),
            pl.BlockSpec((32, 64), lambda w, b, s: (0, 0)),
            pl.BlockSpec((1, 64), lambda w, b, s: (0, 0)),
        ],
        out_specs=[
            pl.BlockSpec((1, 1, _SBLK_MLP, 64), lambda w, b, s: (w, b, s, 0)),
            pl.BlockSpec((1, 1, _SBLK_MLP, 64), lambda w, b, s: (w, b, s, 0)),
            pl.BlockSpec((1, 8, 128), lambda w, b, s: (w, 0, 0)),
        ],
        out_shape=[
            jax.ShapeDtypeStruct((W, B, S, 64), jnp.float32),
            jax.ShapeDtypeStruct((W, B, S, 64), jnp.float32),
            jax.ShapeDtypeStruct((W, 8, 128), jnp.float32),
        ],
        interpret=interpret,
    )(y1, acc1, g1p, be1p, w2T, b2p)


def _p3_body(ymax_ref, ymin_ref, acc2_ref, g2_ref, be2_ref, out_ref,
             *, m_count):
    inv = 1.0 / float(m_count)
    s1 = acc2_ref[0, 0:1, 0:64]
    s2 = acc2_ref[0, 1:2, 0:64]
    mu = s1 * inv
    var = s2 * inv - mu * mu
    a2 = g2_ref[...] * lax.rsqrt(var + 1e-5)
    c2 = be2_ref[...] - mu * a2
    hi = ymax_ref[0, 0]
    lo = ymin_ref[0, 0]
    y = jnp.where(a2 > 0.0, hi * a2 + c2, lo * a2 + c2)
    out_ref[0, 0] = jnp.maximum(y, 0.0)


def _mlp_finalize(ymax, ymin, acc2, g2p, be2p, m_count, interpret=False):
    W, B, S, C = ymax.shape
    grid = (W, B)
    return pl.pallas_call(
        functools.partial(_p3_body, m_count=m_count),
        grid=grid,
        in_specs=[
            pl.BlockSpec((1, 1, S, C), lambda w, b: (w, b, 0, 0)),
            pl.BlockSpec((1, 1, S, C), lambda w, b: (w, b, 0, 0)),
            pl.BlockSpec((1, 8, 128), lambda w, b: (w, 0, 0)),
            pl.BlockSpec((1, 64), lambda w, b: (0, 0)),
            pl.BlockSpec((1, 64), lambda w, b: (0, 0)),
        ],
        out_specs=pl.BlockSpec((1, 1, S, C), lambda w, b: (w, b, 0, 0)),
        out_shape=jax.ShapeDtypeStruct((W, B, S, C), jnp.float32),
        interpret=interpret,
    )(ymax, ymin, acc2, g2p, be2p)


def kernel(xyz, points, center, t_list, W1, b1, g1, be1, W2, b2, g2, be2):
    B, N, _ = xyz.shape
    S = center.shape[1]
    D = points.shape[2]

    xyzT = jnp.transpose(xyz, (0, 2, 1))
    idx1, idx2 = _select_indices(t_list.reshape(B, 1, -1), xyzT, center)

    pad = jnp.zeros((B, N, 32 - 3 - D), jnp.float32)
    table = jnp.concatenate([xyz, points, pad], axis=-1).reshape(B * N, 32)
    idx_all = jnp.concatenate([idx1.reshape(-1), idx2.reshape(-1)])
    n_rows = idx_all.shape[0]

    gathered = _make_sc_gather(n_rows, 32)(table, idx_all)
    g4 = gathered.reshape(2, B, S * _NS, 32)

    w1pT = jnp.pad(W1, ((0, 0), (0, 32 - W1.shape[1]))).T
    b1p = b1[None, :]
    g1p = g1[None, :]
    be1p = be1[None, :]
    w2T = W2.T
    b2p = b2[None, :]
    g2p = g2[None, :]
    be2p = be2[None, :]

    m_count = B * S * _NS
    y1, acc1 = _mlp_pass1(g4, center, w1pT, b1p)
    ymax, ymin, acc2 = _mlp_pass2(y1, acc1, g1p, be1p, w2T, b2p, m_count)
    out = _mlp_finalize(ymax, ymin, acc2, g2p, be2p, m_count)

    res_points = jnp.transpose(out, (1, 0, 3, 2))
    cT = jnp.transpose(center, (0, 2, 1))
    res_xyz = jnp.stack([cT, cT], axis=1)
    return res_xyz, res_points


# speedup vs baseline: 9.2288x; 1.0273x over previous
import functools

import jax
import jax.numpy as jnp
from jax import lax
from jax.experimental import pallas as pl
from jax.experimental.pallas import tpu as pltpu
from jax.experimental.pallas import tpu_sc as plsc

_RADIUS = 2.0
_NS = 32
_W_LIST = ((1.0, 0.2), (0.5, 1.0))
_BIG = 1.0e9

_SBLK_SEL = 128
_SBLK_MLP = 128

_SC_NC = 2
_SC_NSUB = 16
_SC_NW = _SC_NC * _SC_NSUB
_SC_CB = 128


def _sel_body(tl_ref, xyzT_ref, cen_ref, idx1_ref, idx2_ref, *, n, sblk):
    b = pl.program_id(0)
    cen = cen_ref[0]
    pts = xyzT_ref[0]
    tl = tl_ref[0]
    gaps = tl[:, 1:] - tl[:, :-1]
    mt = jnp.max(gaps) ** 2

    cen_b = cen.astype(jnp.bfloat16).astype(jnp.float32)
    pts_b = pts.astype(jnp.bfloat16).astype(jnp.float32)
    sn_xy = jnp.sum(cen[:, 0:2] ** 2, axis=1, keepdims=True)
    dn_xy = jnp.sum(pts[0:2, :] ** 2, axis=0, keepdims=True)
    mm_xy = cen_b[:, 0:1] * pts_b[0:1, :] + cen_b[:, 1:2] * pts_b[1:2, :]
    dxy = -2.0 * mm_xy
    dxy = dxy + sn_xy
    dxy = dxy + dn_xy
    st = cen[:, 2:3]
    dtp = pts[2:3, :]
    dt = -2.0 * (st * dtp)
    dt = dt + st ** 2
    dt = dt + dtp ** 2
    mm_o = (cen_b[:, 0:1] * pts_b[0:1, :] + cen_b[:, 1:2] * pts_b[1:2, :]) \
        + cen_b[:, 2:3] * pts_b[2:3, :]
    dori = -2.0 * mm_o
    dori = dori + jnp.sum(cen ** 2, axis=1, keepdims=True)
    dori = dori + jnp.sum(pts ** 2, axis=0, keepdims=True)
    valid = (dxy <= _RADIUS) & (dt <= mt)
    iota0 = lax.broadcasted_iota(jnp.int32, (sblk, n), 1)

    m0 = jnp.min(dori, axis=1, keepdims=True)
    gf = jnp.min(jnp.where(dori == m0, iota0, n), axis=1, keepdims=True)

    base = b * n
    (wa0, wa1), (wb0, wb1) = _W_LIST
    da0 = jnp.where(valid, dxy * wa0 + dt * wa1, _BIG)
    db0 = jnp.where(valid, dxy * wb0 + dt * wb1, _BIG)
    acc0 = jnp.zeros((sblk, _NS), jnp.int32)

    def body(k, carry):
        d1, d2, a1, a2 = carry
        iota = lax.broadcasted_iota(jnp.int32, (sblk, n), 1)
        kiota = lax.broadcasted_iota(jnp.int32, (sblk, _NS), 1)
        mv1 = jnp.min(d1, axis=1, keepdims=True)
        mv2 = jnp.min(d2, axis=1, keepdims=True)
        ai1 = jnp.min(jnp.where(d1 == mv1, iota, n), axis=1, keepdims=True)
        ai2 = jnp.min(jnp.where(d2 == mv2, iota, n), axis=1, keepdims=True)
        sel1 = jnp.where(mv1 >= _BIG, gf, ai1)
        sel2 = jnp.where(mv2 >= _BIG, gf, ai2)
        a1 = jnp.where(kiota == k, sel1 + base, a1)
        a2 = jnp.where(kiota == k, sel2 + base, a2)
        d1 = jnp.where(iota == ai1, _BIG, d1)
        d2 = jnp.where(iota == ai2, _BIG, d2)
        return d1, d2, a1, a2

    _, _, acc1, acc2 = lax.fori_loop(0, _NS, body, (da0, db0, acc0, acc0))
    idx1_ref[0] = acc1
    idx2_ref[0] = acc2


def _select_indices(t_list, xyzT, center, interpret=False):
    B, _, N = xyzT.shape
    S = center.shape[1]
    grid = (B, S // _SBLK_SEL)
    out_shape = jax.ShapeDtypeStruct((B, S, _NS), jnp.int32)
    return pl.pallas_call(
        functools.partial(_sel_body, n=N, sblk=_SBLK_SEL),
        grid=grid,
        in_specs=[
            pl.BlockSpec((1, 1, 16), lambda b, s: (b, 0, 0)),
            pl.BlockSpec((1, 3, N), lambda b, s: (b, 0, 0)),
            pl.BlockSpec((1, _SBLK_SEL, 3), lambda b, s: (b, s, 0)),
        ],
        out_specs=[
            pl.BlockSpec((1, _SBLK_SEL, _NS), lambda b, s: (b, s, 0)),
            pl.BlockSpec((1, _SBLK_SEL, _NS), lambda b, s: (b, s, 0)),
        ],
        out_shape=[out_shape, out_shape],
        interpret=interpret,
    )(t_list, xyzT, center)


def _make_sc_gather(n_rows, dp):
    b_per_w = n_rows // _SC_NW
    n_chunks = b_per_w // _SC_CB
    mesh = plsc.VectorSubcoreMesh(core_axis_name="c", subcore_axis_name="s")

    @functools.partial(
        pl.kernel,
        mesh=mesh,
        compiler_params=pltpu.CompilerParams(use_tc_tiling_on_sc=False),
        out_type=jax.ShapeDtypeStruct((n_rows, dp), jnp.float32),
        scratch_types=[
            pltpu.VMEM((_SC_CB,), jnp.int32),
            pltpu.VMEM((_SC_CB, dp), jnp.float32),
            pltpu.SemaphoreType.DMA,
        ],
    )
    def gather_k(table_hbm, idx_hbm, out_hbm, idx_v, rows_v, sem):
        wid = lax.axis_index("s") * _SC_NC + lax.axis_index("c")
        base_w = wid * b_per_w

        def chunk(i, _):
            base = base_w + i * _SC_CB
            pltpu.sync_copy(idx_hbm.at[pl.ds(base, _SC_CB)], idx_v)
            pltpu.async_copy(table_hbm.at[idx_v], rows_v, sem).wait()
            pltpu.sync_copy(rows_v, out_hbm.at[pl.ds(base, _SC_CB)])
            return 0

        lax.fori_loop(0, n_chunks, chunk, 0)

    return gather_k


def _p1_body(g_ref, cen_ref, w1_ref, b1_ref, y1_ref, acc_ref, *, sblk):
    first = (pl.program_id(1) == 0) & (pl.program_id(2) == 0)
    x = g_ref[0, 0]
    cen = cen_ref[0]
    sub = jnp.concatenate(
        [cen, jnp.zeros((sblk, 29), jnp.float32)], axis=1)
    x3 = x.reshape(sblk, _NS, 32) - sub[:, None, :]
    xf = x3.reshape(sblk * _NS, 32)
    y1 = jnp.dot(xf, w1_ref[...], preferred_element_type=jnp.float32)
    y1 = y1 + b1_ref[...]
    y1_ref[0, 0] = y1

    s1 = jnp.sum(y1, axis=0, keepdims=True)
    s2 = jnp.sum(y1 * y1, axis=0, keepdims=True)
    z96 = jnp.zeros((1, 96), jnp.float32)
    row0 = jnp.concatenate([s1, z96], axis=1)
    row1 = jnp.concatenate([s2, z96], axis=1)
    blk = jnp.concatenate(
        [row0, row1, jnp.zeros((6, 128), jnp.float32)], axis=0)

    @pl.when(first)
    def _():
        acc_ref[...] = jnp.zeros_like(acc_ref)

    acc_ref[...] = acc_ref[...] + blk[None]


def _mlp_pass1(g4, center, w1pT, b1p, interpret=False):
    W, B, SN, _ = g4.shape
    S = center.shape[1]
    grid = (W, B, S // _SBLK_MLP)
    return pl.pallas_call(
        functools.partial(_p1_body, sblk=_SBLK_MLP),
        grid=grid,
        in_specs=[
            pl.BlockSpec((1, 1, _SBLK_MLP * _NS, 32), lambda w, b, s: (w, b, s, 0)),
            pl.BlockSpec((1, _SBLK_MLP, 3), lambda w, b, s: (b, s, 0)),
            pl.BlockSpec((32, 32), lambda w, b, s: (0, 0)),
            pl.BlockSpec((1, 32), lambda w, b, s: (0, 0)),
        ],
        out_specs=[
            pl.BlockSpec((1, 1, _SBLK_MLP * _NS, 32), lambda w, b, s: (w, b, s, 0)),
            pl.BlockSpec((1, 8, 128), lambda w, b, s: (w, 0, 0)),
        ],
        out_shape=[
            jax.ShapeDtypeStruct((W, B, SN, 32), jnp.float32),
            jax.ShapeDtypeStruct((W, 8, 128), jnp.float32),
        ],
        interpret=interpret,
    )(g4, center, w1pT, b1p)


def _p2_body(y1_ref, acc1_ref, g1_ref, be1_ref, w2_ref, b2_ref,
             ymax_ref, ymin_ref, acc2_ref, *, sblk, m_count):
    first = (pl.program_id(1) == 0) & (pl.program_id(2) == 0)
    inv = 1.0 / float(m_count)
    s1 = acc1_ref[0, 0:1, 0:32]
    s2 = acc1_ref[0, 1:2, 0:32]
    mu = s1 * inv
    var = s2 * inv - mu * mu
    a1 = g1_ref[...] * lax.rsqrt(var + 1e-5)
    c1 = be1_ref[...] - mu * a1

    y1 = y1_ref[0, 0]
    r = jnp.maximum(y1 * a1 + c1, 0.0)
    y2 = jnp.dot(r, w2_ref[...], preferred_element_type=jnp.float32)
    y2 = y2 + b2_ref[...]

    y3 = y2.reshape(sblk, _NS, 64)
    ymax_ref[0, 0] = jnp.max(y3, axis=1)
    ymin_ref[0, 0] = jnp.min(y3, axis=1)

    t1 = jnp.sum(y2, axis=0, keepdims=True)
    t2 = jnp.sum(y2 * y2, axis=0, keepdims=True)
    z64 = jnp.zeros((1, 64), jnp.float32)
    row0 = jnp.concatenate([t1, z64], axis=1)
    row1 = jnp.concatenate([t2, z64], axis=1)
    blk = jnp.concatenate(
        [row0, row1, jnp.zeros((6, 128), jnp.float32)], axis=0)

    @pl.when(first)
    def _():
        acc2_ref[...] = jnp.zeros_like(acc2_ref)

    acc2_ref[...] = acc2_ref[...] + blk[None]


def _mlp_pass2(y1, acc1, g1p, be1p, w2T, b2p, m_count, interpret=False):
    W, B, SN, _ = y1.shape
    S = SN // _NS
    grid = (W, B, S // _SBLK_MLP)
    return pl.pallas_call(
        functools.partial(_p2_body, sblk=_SBLK_MLP, m_count=m_count),
        grid=grid,
        in_specs=[
            pl.BlockSpec((1, 1, _SBLK_MLP * _NS, 32), lambda w, b, s: (w, b, s, 0)),
            pl.BlockSpec((1, 8, 128), lambda w, b, s: (w, 0, 0)),
            pl.BlockSpec((1, 32), lambda w, b, s: (0, 0)),
            pl.BlockSpec((1, 32), lambda w, b, s: (0, 0)),
            pl.BlockSpec((32, 64), lambda w, b, s: (0, 0)),
            pl.BlockSpec((1, 64), lambda w, b, s: (0, 0)),
        ],
        out_specs=[
            pl.BlockSpec((1, 1, _SBLK_MLP, 64), lambda w, b, s: (w, b, s, 0)),
            pl.BlockSpec((1, 1, _SBLK_MLP, 64), lambda w, b, s: (w, b, s, 0)),
            pl.BlockSpec((1, 8, 128), lambda w, b, s: (w, 0, 0)),
        ],
        out_shape=[
            jax.ShapeDtypeStruct((W, B, S, 64), jnp.float32),
            jax.ShapeDtypeStruct((W, B, S, 64), jnp.float32),
            jax.ShapeDtypeStruct((W, 8, 128), jnp.float32),
        ],
        interpret=interpret,
    )(y1, acc1, g1p, be1p, w2T, b2p)


def _p3_body(ymax_ref, ymin_ref, acc2_ref, g2_ref, be2_ref, out_ref,
             *, m_count):
    inv = 1.0 / float(m_count)
    s1 = acc2_ref[0, 0:1, 0:64]
    s2 = acc2_ref[0, 1:2, 0:64]
    mu = s1 * inv
    var = s2 * inv - mu * mu
    a2 = g2_ref[...] * lax.rsqrt(var + 1e-5)
    c2 = be2_ref[...] - mu * a2
    hi = ymax_ref[0, 0]
    lo = ymin_ref[0, 0]
    y = jnp.where(a2 > 0.0, hi * a2 + c2, lo * a2 + c2)
    out_ref[0, 0] = jnp.maximum(y, 0.0)


def _mlp_finalize(ymax, ymin, acc2, g2p, be2p, m_count, interpret=False):
    W, B, S, C = ymax.shape
    grid = (W, B)
    return pl.pallas_call(
        functools.partial(_p3_body, m_count=m_count),
        grid=grid,
        in_specs=[
            pl.BlockSpec((1, 1, S, C), lambda w, b: (w, b, 0, 0)),
            pl.BlockSpec((1, 1, S, C), lambda w, b: (w, b, 0, 0)),
            pl.BlockSpec((1, 8, 128), lambda w, b: (w, 0, 0)),
            pl.BlockSpec((1, 64), lambda w, b: (0, 0)),
            pl.BlockSpec((1, 64), lambda w, b: (0, 0)),
        ],
        out_specs=pl.BlockSpec((1, 1, S, C), lambda w, b: (w, b, 0, 0)),
        out_shape=jax.ShapeDtypeStruct((W, B, S, C), jnp.float32),
        interpret=interpret,
    )(ymax, ymin, acc2, g2p, be2p)


def kernel(xyz, points, center, t_list, W1, b1, g1, be1, W2, b2, g2, be2):
    B, N, _ = xyz.shape
    S = center.shape[1]
    D = points.shape[2]

    xyzT = jnp.transpose(xyz, (0, 2, 1))
    idx1, idx2 = _select_indices(t_list.reshape(B, 1, -1), xyzT, center)

    pad = jnp.zeros((B, N, 32 - 3 - D), jnp.float32)
    table = jnp.concatenate([xyz, points, pad], axis=-1).reshape(B * N, 32)
    idx_all = jnp.concatenate([idx1.reshape(-1), idx2.reshape(-1)])
    n_rows = idx_all.shape[0]

    gathered = _make_sc_gather(n_rows, 32)(table, idx_all)
    g4 = gathered.reshape(2, B, S * _NS, 32)

    w1pT = jnp.pad(W1, ((0, 0), (0, 32 - W1.shape[1]))).T
    b1p = b1[None, :]
    g1p = g1[None, :]
    be1p = be1[None, :]
    w2T = W2.T
    b2p = b2[None, :]
    g2p = g2[None, :]
    be2p = be2[None, :]

    m_count = B * S * _NS
    y1, acc1 = _mlp_pass1(g4, center, w1pT, b1p)
    ymax, ymin, acc2 = _mlp_pass2(y1, acc1, g1p, be1p, w2T, b2p, m_count)
    out = _mlp_finalize(ymax, ymin, acc2, g2p, be2p, m_count)

    res_points = jnp.transpose(out, (1, 0, 3, 2))
    cT = jnp.transpose(center, (0, 2, 1))
    res_xyz = jnp.stack([cT, cT], axis=1)
    return res_xyz, res_points
